# baseline (speedup 1.0000x reference)
import functools

import jax
import jax.numpy as jnp
from jax import lax
from jax.experimental import pallas as pl
from jax.experimental.pallas import tpu as pltpu
from jax.experimental.pallas import tpu_sc as plsc

_NUM_CORES = 2
_NUM_SUBCORES = 16
_NW = _NUM_CORES * _NUM_SUBCORES
_LANES = 16


def _make_agg_kernel(S, DEG, D, C):
    assert S % C == 0
    n_chunks = S // C
    assert n_chunks >= 2 * _NW
    max_iters = (n_chunks + _NW - 1) // _NW
    half_iters = (max_iters + 1) // 2
    groups = D // _LANES
    inv_deg = 1.0 / DEG

    mesh = plsc.VectorSubcoreMesh(core_axis_name="c", subcore_axis_name="s")

    @functools.partial(
        pl.kernel,
        out_type=jax.ShapeDtypeStruct((S, 2 * D), jnp.float32),
        mesh=mesh,
        scratch_types=[
            pltpu.VMEM((2, C, DEG, D), jnp.float32),
            pltpu.VMEM((2, C, 2 * D), jnp.float32),
            pltpu.SemaphoreType.DMA,
            pltpu.SemaphoreType.DMA,
            pltpu.SemaphoreType.DMA,
            pltpu.SemaphoreType.DMA,
        ],
    )
    def agg(mb_hbm, out_hbm, buf, obuf, isem0, isem1, osem0, osem1):
        wid = lax.axis_index("s") * _NUM_CORES + lax.axis_index("c")
        my = (n_chunks - wid + _NW - 1) // _NW
        isems = (isem0, isem1)
        osems = (osem0, osem1)

        def in_copy(k, p):
            base = (k * _NW + wid) * C
            return pltpu.make_async_copy(
                mb_hbm.at[pl.ds(base, C)], buf.at[p], isems[p]
            )

        def out_copy(k, p):
            base = (k * _NW + wid) * C
            return pltpu.make_async_copy(
                obuf.at[p], out_hbm.at[pl.ds(base, C)], osems[p]
            )

        in_copy(0, 0).start()

        def body(kk, _):
            for p in range(2):
                k = kk * 2 + p
                np_ = 1 - p

                @pl.when(k + 1 < my)
                def _():
                    in_copy(k + 1, np_).start()

                @pl.when(k < my)
                def _():
                    in_copy(k, p).wait()

                    @pl.when(k >= 2)
                    def _():
                        out_copy(k - 2, p).wait()

                    def node_body(n, carry):
                        for g in range(groups):
                            col = pl.ds(g * _LANES, _LANES)
                            s = buf[p, n, 0, col]
                            m = s
                            for r in range(1, DEG):
                                v = buf[p, n, r, col]
                                s = s + v
                                m = jnp.maximum(m, v)
                            obuf[p, n, col] = s * inv_deg
                            obuf[p, n, pl.ds(D + g * _LANES, _LANES)] = m
                        return carry

                    lax.fori_loop(0, C, node_body, 0)
                    out_copy(k, p).start()

            return 0

        lax.fori_loop(0, half_iters, body, 0)

        out_copy(0, 0).wait()
        out_copy(0, 1).wait()

    return agg


def _make_fused_body(DEG, D):
    def fused_body(mb_ref, wm_ref, wx_ref, b_ref, o_ref):
        s = mb_ref[:, pl.ds(0, D)]
        m = s
        for r in range(1, DEG):
            v = mb_ref[:, pl.ds(r * D, D)]
            s = s + v
            m = jnp.maximum(m, v)
        o_ref[...] = (
            jnp.dot(s * (1.0 / DEG), wm_ref[...], preferred_element_type=jnp.float32)
            + jnp.dot(m, wx_ref[...], preferred_element_type=jnp.float32)
            + b_ref[...]
        )

    return fused_body


def _mm_body(a_ref, wt_ref, b_ref, o_ref):
    o_ref[...] = (
        jnp.dot(a_ref[...], wt_ref[...], preferred_element_type=jnp.float32)
        + b_ref[...]
    )


def kernel(mailbox, W, b):
    N, DEG, D = mailbox.shape
    S = 4000
    C = 8
    Bn = 80
    Bm = 1000

    Wt = W.T
    wm = Wt[:D]
    wx = Wt[D:]
    b2 = b.reshape(1, D)

    agg = _make_agg_kernel(S, DEG, D, C)(mailbox)

    mb2 = mailbox.reshape(N, DEG * D)
    s_blocks = S // Bn
    out_tail = pl.pallas_call(
        _make_fused_body(DEG, D),
        grid=((N - S) // Bn,),
        in_specs=[
            pl.BlockSpec((Bn, DEG * D), lambda i: (i + s_blocks, 0)),
            pl.BlockSpec((D, D), lambda i: (0, 0)),
            pl.BlockSpec((D, D), lambda i: (0, 0)),
            pl.BlockSpec((1, D), lambda i: (0, 0)),
        ],
        out_specs=pl.BlockSpec((Bn, D), lambda i: (i, 0)),
        out_shape=jax.ShapeDtypeStruct((N - S, D), jnp.float32),
    )(mb2, wm, wx, b2)

    out_head = pl.pallas_call(
        _mm_body,
        grid=(S // Bm,),
        in_specs=[
            pl.BlockSpec((Bm, 2 * D), lambda i: (i, 0)),
            pl.BlockSpec((2 * D, D), lambda i: (0, 0)),
            pl.BlockSpec((1, D), lambda i: (0, 0)),
        ],
        out_specs=pl.BlockSpec((Bm, D), lambda i: (i, 0)),
        out_shape=jax.ShapeDtypeStruct((S, D), jnp.float32),
    )(agg, Wt, b2)

    return jnp.concatenate([out_head, out_tail], axis=0)


# speedup vs baseline: 2.5863x; 2.5863x over previous
import functools

import jax
import jax.numpy as jnp
from jax import lax
from jax.experimental import pallas as pl
from jax.experimental.pallas import tpu as pltpu
from jax.experimental.pallas import tpu_sc as plsc

_NUM_CORES = 2
_NUM_SUBCORES = 16
_NW = _NUM_CORES * _NUM_SUBCORES
_LANES = 16


def _make_agg_kernel(S, DEG, D, C):
    assert S % C == 0
    n_chunks = S // C
    assert n_chunks >= 2 * _NW
    max_iters = (n_chunks + _NW - 1) // _NW
    half_iters = (max_iters + 1) // 2
    groups = D // _LANES
    inv_deg = 1.0 / DEG

    mesh = plsc.VectorSubcoreMesh(core_axis_name="c", subcore_axis_name="s")

    @functools.partial(
        pl.kernel,
        out_type=jax.ShapeDtypeStruct((S, 2 * D), jnp.float32),
        mesh=mesh,
        scratch_types=[
            pltpu.VMEM((2, C, DEG, D), jnp.float32),
            pltpu.VMEM((2, C, 2 * D), jnp.float32),
            pltpu.SemaphoreType.DMA,
            pltpu.SemaphoreType.DMA,
            pltpu.SemaphoreType.DMA,
            pltpu.SemaphoreType.DMA,
        ],
    )
    def agg(mb_hbm, out_hbm, buf, obuf, isem0, isem1, osem0, osem1):
        wid = lax.axis_index("s") * _NUM_CORES + lax.axis_index("c")
        my = (n_chunks - wid + _NW - 1) // _NW
        isems = (isem0, isem1)
        osems = (osem0, osem1)

        def in_copy(k, p):
            base = (k * _NW + wid) * C
            return pltpu.make_async_copy(
                mb_hbm.at[pl.ds(base, C)], buf.at[p], isems[p]
            )

        def out_copy(k, p):
            base = (k * _NW + wid) * C
            return pltpu.make_async_copy(
                obuf.at[p], out_hbm.at[pl.ds(base, C)], osems[p]
            )

        in_copy(0, 0).start()

        def body(kk, _):
            for p in range(2):
                k = kk * 2 + p
                np_ = 1 - p

                @pl.when(k + 1 < my)
                def _():
                    in_copy(k + 1, np_).start()

                @pl.when(k < my)
                def _():
                    in_copy(k, p).wait()

                    @pl.when(k >= 2)
                    def _():
                        out_copy(k - 2, p).wait()

                    def node_body(n, carry):
                        for g in range(groups):
                            col = pl.ds(g * _LANES, _LANES)
                            s = buf[p, n, 0, col]
                            m = s
                            for r in range(1, DEG):
                                v = buf[p, n, r, col]
                                s = s + v
                                m = jnp.maximum(m, v)
                            obuf[p, n, col] = s * inv_deg
                            obuf[p, n, pl.ds(D + g * _LANES, _LANES)] = m
                        return carry

                    lax.fori_loop(0, C, node_body, 0)
                    out_copy(k, p).start()

            return 0

        lax.fori_loop(0, half_iters, body, 0)

        out_copy(0, 0).wait()
        out_copy(0, 1).wait()

    return agg


def _make_fused_body(DEG, D):
    def fused_body(mb_ref, wm_ref, wx_ref, b_ref, o_ref):
        blk = mb_ref[...]
        s = jnp.sum(blk, axis=1)
        m = jnp.max(blk, axis=1)
        o_ref[...] = (
            jnp.dot(s * (1.0 / DEG), wm_ref[...], preferred_element_type=jnp.float32)
            + jnp.dot(m, wx_ref[...], preferred_element_type=jnp.float32)
            + b_ref[...]
        )

    return fused_body


def _mm_body(a_ref, wt_ref, b_ref, o_ref):
    o_ref[...] = (
        jnp.dot(a_ref[...], wt_ref[...], preferred_element_type=jnp.float32)
        + b_ref[...]
    )


def kernel(mailbox, W, b):
    N, DEG, D = mailbox.shape
    S = 4000
    C = 8
    Bn = 1000
    Bm = 1000

    Wt = W.T
    wm = Wt[:D]
    wx = Wt[D:]
    b2 = b.reshape(1, D)

    agg = _make_agg_kernel(S, DEG, D, C)(mailbox)

    s_blocks = S // Bn
    out_tail = pl.pallas_call(
        _make_fused_body(DEG, D),
        grid=((N - S) // Bn,),
        in_specs=[
            pl.BlockSpec((Bn, DEG, D), lambda i: (i + s_blocks, 0, 0)),
            pl.BlockSpec((D, D), lambda i: (0, 0)),
            pl.BlockSpec((D, D), lambda i: (0, 0)),
            pl.BlockSpec((1, D), lambda i: (0, 0)),
        ],
        out_specs=pl.BlockSpec((Bn, D), lambda i: (i, 0)),
        out_shape=jax.ShapeDtypeStruct((N - S, D), jnp.float32),
    )(mailbox, wm, wx, b2)

    out_head = pl.pallas_call(
        _mm_body,
        grid=(S // Bm,),
        in_specs=[
            pl.BlockSpec((Bm, 2 * D), lambda i: (i, 0)),
            pl.BlockSpec((2 * D, D), lambda i: (0, 0)),
            pl.BlockSpec((1, D), lambda i: (0, 0)),
        ],
        out_specs=pl.BlockSpec((Bm, D), lambda i: (i, 0)),
        out_shape=jax.ShapeDtypeStruct((S, D), jnp.float32),
    )(agg, Wt, b2)

    return jnp.concatenate([out_head, out_tail], axis=0)


# speedup vs baseline: 2.7260x; 1.0540x over previous
import functools

import jax
import jax.numpy as jnp
from jax import lax
from jax.experimental import pallas as pl
from jax.experimental.pallas import tpu as pltpu
from jax.experimental.pallas import tpu_sc as plsc

_NUM_CORES = 2
_NUM_SUBCORES = 16
_NW = _NUM_CORES * _NUM_SUBCORES
_LANES = 16


def _make_agg_kernel(S, DEG, D, C):
    assert S % C == 0
    n_chunks = S // C
    assert n_chunks >= 2 * _NW
    max_iters = (n_chunks + _NW - 1) // _NW
    half_iters = (max_iters + 1) // 2
    groups = D // _LANES
    inv_deg = 1.0 / DEG

    mesh = plsc.VectorSubcoreMesh(core_axis_name="c", subcore_axis_name="s")

    @functools.partial(
        pl.kernel,
        out_type=jax.ShapeDtypeStruct((S, 2 * D), jnp.float32),
        mesh=mesh,
        scratch_types=[
            pltpu.VMEM((2, C, DEG, D), jnp.float32),
            pltpu.VMEM((2, C, 2 * D), jnp.float32),
            pltpu.SemaphoreType.DMA,
            pltpu.SemaphoreType.DMA,
            pltpu.SemaphoreType.DMA,
            pltpu.SemaphoreType.DMA,
        ],
    )
    def agg(mb_hbm, out_hbm, buf, obuf, isem0, isem1, osem0, osem1):
        wid = lax.axis_index("s") * _NUM_CORES + lax.axis_index("c")
        my = (n_chunks - wid + _NW - 1) // _NW
        isems = (isem0, isem1)
        osems = (osem0, osem1)

        def in_copy(k, p):
            base = (k * _NW + wid) * C
            return pltpu.make_async_copy(
                mb_hbm.at[pl.ds(base, C)], buf.at[p], isems[p]
            )

        def out_copy(k, p):
            base = (k * _NW + wid) * C
            return pltpu.make_async_copy(
                obuf.at[p], out_hbm.at[pl.ds(base, C)], osems[p]
            )

        in_copy(0, 0).start()

        def body(kk, _):
            for p in range(2):
                k = kk * 2 + p
                np_ = 1 - p

                @pl.when(k + 1 < my)
                def _():
                    in_copy(k + 1, np_).start()

                @pl.when(k < my)
                def _():
                    in_copy(k, p).wait()

                    @pl.when(k >= 2)
                    def _():
                        out_copy(k - 2, p).wait()

                    def node_body(n, carry):
                        for g in range(groups):
                            col = pl.ds(g * _LANES, _LANES)
                            s = buf[p, n, 0, col]
                            m = s
                            for r in range(1, DEG):
                                v = buf[p, n, r, col]
                                s = s + v
                                m = jnp.maximum(m, v)
                            obuf[p, n, col] = s * inv_deg
                            obuf[p, n, pl.ds(D + g * _LANES, _LANES)] = m
                        return carry

                    lax.fori_loop(0, C, node_body, 0)
                    out_copy(k, p).start()

            return 0

        lax.fori_loop(0, half_iters, body, 0)

        out_copy(0, 0).wait()
        out_copy(0, 1).wait()

    return agg


def _make_fused_body(DEG, D):
    def fused_body(mb_ref, w_ref, b_ref, o_ref):
        blk = mb_ref[...]
        s = jnp.sum(blk, axis=1)
        m = jnp.max(blk, axis=1)
        dn = (((1,), (1,)), ((), ()))
        o_ref[...] = (
            lax.dot_general(
                s * (1.0 / DEG),
                w_ref[:, :D],
                dn,
                preferred_element_type=jnp.float32,
            )
            + lax.dot_general(
                m, w_ref[:, D:], dn, preferred_element_type=jnp.float32
            )
            + b_ref[...]
        )

    return fused_body


def _mm_body(a_ref, w_ref, b_ref, _aliased_ref, o_ref):
    dn = (((1,), (1,)), ((), ()))
    o_ref[...] = (
        lax.dot_general(
            a_ref[...], w_ref[...], dn, preferred_element_type=jnp.float32
        )
        + b_ref[...]
    )


def kernel(mailbox, W, b):
    N, DEG, D = mailbox.shape
    S = 4000
    C = 8
    Bn = 1000
    Bm = 1000

    b2 = b.reshape(1, D)

    agg = _make_agg_kernel(S, DEG, D, C)(mailbox)

    s_blocks = S // Bn
    out_tail = pl.pallas_call(
        _make_fused_body(DEG, D),
        grid=((N - S) // Bn,),
        in_specs=[
            pl.BlockSpec((Bn, DEG, D), lambda i: (i + s_blocks, 0, 0)),
            pl.BlockSpec((D, 2 * D), lambda i: (0, 0)),
            pl.BlockSpec((1, D), lambda i: (0, 0)),
        ],
        out_specs=pl.BlockSpec((Bn, D), lambda i: (i + s_blocks, 0)),
        out_shape=jax.ShapeDtypeStruct((N, D), jnp.float32),
    )(mailbox, W, b2)

    out = pl.pallas_call(
        _mm_body,
        grid=(S // Bm,),
        in_specs=[
            pl.BlockSpec((Bm, 2 * D), lambda i: (i, 0)),
            pl.BlockSpec((D, 2 * D), lambda i: (0, 0)),
            pl.BlockSpec((1, D), lambda i: (0, 0)),
            pl.BlockSpec(memory_space=pl.ANY),
        ],
        out_specs=pl.BlockSpec((Bm, D), lambda i: (i, 0)),
        out_shape=jax.ShapeDtypeStruct((N, D), jnp.float32),
        input_output_aliases={3: 0},
    )(agg, W, b2, out_tail)

    return out


# speedup vs baseline: 2.9382x; 1.0779x over previous
import functools

import jax
import jax.numpy as jnp
from jax import lax
from jax.experimental import pallas as pl
from jax.experimental.pallas import tpu as pltpu
from jax.experimental.pallas import tpu_sc as plsc

_NUM_CORES = 2
_NUM_SUBCORES = 16
_NW = _NUM_CORES * _NUM_SUBCORES
_LANES = 16


def _make_agg_kernel(S, DEG, D, C):
    assert S % C == 0
    n_chunks = S // C
    assert n_chunks >= 2 * _NW
    max_iters = (n_chunks + _NW - 1) // _NW
    half_iters = (max_iters + 1) // 2
    groups = D // _LANES
    inv_deg = 1.0 / DEG

    mesh = plsc.VectorSubcoreMesh(core_axis_name="c", subcore_axis_name="s")

    @functools.partial(
        pl.kernel,
        out_type=jax.ShapeDtypeStruct((S, 2 * D), jnp.float32),
        mesh=mesh,
        scratch_types=[
            pltpu.VMEM((2, C, DEG, D), jnp.float32),
            pltpu.VMEM((2, C, 2 * D), jnp.float32),
            pltpu.SemaphoreType.DMA,
            pltpu.SemaphoreType.DMA,
            pltpu.SemaphoreType.DMA,
            pltpu.SemaphoreType.DMA,
        ],
    )
    def agg(mb_hbm, out_hbm, buf, obuf, isem0, isem1, osem0, osem1):
        wid = lax.axis_index("s") * _NUM_CORES + lax.axis_index("c")
        my = (n_chunks - wid + _NW - 1) // _NW
        isems = (isem0, isem1)
        osems = (osem0, osem1)

        def in_copy(k, p):
            base = (k * _NW + wid) * C
            return pltpu.make_async_copy(
                mb_hbm.at[pl.ds(base, C)], buf.at[p], isems[p]
            )

        def out_copy(k, p):
            base = (k * _NW + wid) * C
            return pltpu.make_async_copy(
                obuf.at[p], out_hbm.at[pl.ds(base, C)], osems[p]
            )

        in_copy(0, 0).start()

        def body(kk, _):
            for p in range(2):
                k = kk * 2 + p
                np_ = 1 - p

                @pl.when(k + 1 < my)
                def _():
                    in_copy(k + 1, np_).start()

                @pl.when(k < my)
                def _():
                    in_copy(k, p).wait()

                    @pl.when(k >= 2)
                    def _():
                        out_copy(k - 2, p).wait()

                    def node_body(n, carry):
                        for g in range(groups):
                            col = pl.ds(g * _LANES, _LANES)
                            s = buf[p, n, 0, col]
                            m = s
                            for r in range(1, DEG):
                                v = buf[p, n, r, col]
                                s = s + v
                                m = jnp.maximum(m, v)
                            obuf[p, n, col] = s * inv_deg
                            obuf[p, n, pl.ds(D + g * _LANES, _LANES)] = m
                        return carry

                    lax.fori_loop(0, C, node_body, 0)
                    out_copy(k, p).start()

            return 0

        lax.fori_loop(0, half_iters, body, 0)

        out_copy(0, 0).wait()
        out_copy(0, 1).wait()

    return agg


def _make_fused_body(DEG, D):
    def fused_body(mb_ref, w_ref, b_ref, o_ref):
        blk = mb_ref[...]
        s = jnp.sum(blk, axis=1)
        m = jnp.max(blk, axis=1)
        dn = (((1,), (1,)), ((), ()))
        o_ref[...] = (
            lax.dot_general(
                s * (1.0 / DEG),
                w_ref[:, :D],
                dn,
                preferred_element_type=jnp.float32,
            )
            + lax.dot_general(
                m, w_ref[:, D:], dn, preferred_element_type=jnp.float32
            )
            + b_ref[...]
        )

    return fused_body


def _mm_body(a_ref, w_ref, b_ref, _aliased_ref, o_ref):
    dn = (((1,), (1,)), ((), ()))
    o_ref[...] = (
        lax.dot_general(
            a_ref[...], w_ref[...], dn, preferred_element_type=jnp.float32
        )
        + b_ref[...]
    )


def kernel(mailbox, W, b):
    N, DEG, D = mailbox.shape
    S = 2000
    C = 8
    Bn = 1000
    Bm = 1000

    b2 = b.reshape(1, D)

    agg = _make_agg_kernel(S, DEG, D, C)(mailbox)

    s_blocks = S // Bn
    out_tail = pl.pallas_call(
        _make_fused_body(DEG, D),
        grid=((N - S) // Bn,),
        in_specs=[
            pl.BlockSpec((Bn, DEG, D), lambda i: (i + s_blocks, 0, 0)),
            pl.BlockSpec((D, 2 * D), lambda i: (0, 0)),
            pl.BlockSpec((1, D), lambda i: (0, 0)),
        ],
        out_specs=pl.BlockSpec((Bn, D), lambda i: (i + s_blocks, 0)),
        out_shape=jax.ShapeDtypeStruct((N, D), jnp.float32),
    )(mailbox, W, b2)

    out = pl.pallas_call(
        _mm_body,
        grid=(S // Bm,),
        in_specs=[
            pl.BlockSpec((Bm, 2 * D), lambda i: (i, 0)),
            pl.BlockSpec((D, 2 * D), lambda i: (0, 0)),
            pl.BlockSpec((1, D), lambda i: (0, 0)),
            pl.BlockSpec(memory_space=pl.ANY),
        ],
        out_specs=pl.BlockSpec((Bm, D), lambda i: (i, 0)),
        out_shape=jax.ShapeDtypeStruct((N, D), jnp.float32),
        input_output_aliases={3: 0},
    )(agg, W, b2, out_tail)

    return out


# speedup vs baseline: 2.9459x; 1.0026x over previous
import functools

import jax
import jax.numpy as jnp
from jax import lax
from jax.experimental import pallas as pl
from jax.experimental.pallas import tpu as pltpu
from jax.experimental.pallas import tpu_sc as plsc

_NUM_CORES = 2
_NUM_SUBCORES = 16
_NW = _NUM_CORES * _NUM_SUBCORES
_LANES = 16


def _make_agg_kernel(S, DEG, D, C):
    assert S % C == 0
    n_chunks = S // C
    assert n_chunks >= 2 * _NW
    max_iters = (n_chunks + _NW - 1) // _NW
    half_iters = (max_iters + 1) // 2
    groups = D // _LANES
    inv_deg = 1.0 / DEG

    mesh = plsc.VectorSubcoreMesh(core_axis_name="c", subcore_axis_name="s")

    @functools.partial(
        pl.kernel,
        out_type=jax.ShapeDtypeStruct((S, 2 * D), jnp.float32),
        mesh=mesh,
        scratch_types=[
            pltpu.VMEM((2, C, DEG, D), jnp.float32),
            pltpu.VMEM((2, C, 2 * D), jnp.float32),
            pltpu.SemaphoreType.DMA,
            pltpu.SemaphoreType.DMA,
            pltpu.SemaphoreType.DMA,
            pltpu.SemaphoreType.DMA,
        ],
    )
    def agg(mb_hbm, out_hbm, buf, obuf, isem0, isem1, osem0, osem1):
        wid = lax.axis_index("s") * _NUM_CORES + lax.axis_index("c")
        my = (n_chunks - wid + _NW - 1) // _NW
        isems = (isem0, isem1)
        osems = (osem0, osem1)

        def in_copy(k, p):
            base = (k * _NW + wid) * C
            return pltpu.make_async_copy(
                mb_hbm.at[pl.ds(base, C)], buf.at[p], isems[p]
            )

        def out_copy(k, p):
            base = (k * _NW + wid) * C
            return pltpu.make_async_copy(
                obuf.at[p], out_hbm.at[pl.ds(base, C)], osems[p]
            )

        in_copy(0, 0).start()

        def body(kk, _):
            for p in range(2):
                k = kk * 2 + p
                np_ = 1 - p

                @pl.when(k + 1 < my)
                def _():
                    in_copy(k + 1, np_).start()

                @pl.when(k < my)
                def _():
                    in_copy(k, p).wait()

                    @pl.when(k >= 2)
                    def _():
                        out_copy(k - 2, p).wait()

                    def group_body(t, carry):
                        n = t // groups
                        g = t % groups
                        col = pl.ds(g * _LANES, _LANES)
                        s = buf[p, n, 0, col]
                        m = s
                        for r in range(1, DEG):
                            v = buf[p, n, r, col]
                            s = s + v
                            m = jnp.maximum(m, v)
                        obuf[p, n, col] = s * inv_deg
                        obuf[p, n, pl.ds(D + g * _LANES, _LANES)] = m
                        return carry

                    lax.fori_loop(0, C * groups, group_body, 0)
                    out_copy(k, p).start()

            return 0

        lax.fori_loop(0, half_iters, body, 0)

        out_copy(0, 0).wait()
        out_copy(0, 1).wait()

    return agg


def _make_fused_body(DEG, D):
    def fused_body(mb_ref, w_ref, b_ref, o_ref):
        blk = mb_ref[...]
        s = jnp.sum(blk, axis=1)
        m = jnp.max(blk, axis=1)
        dn = (((1,), (1,)), ((), ()))
        o_ref[...] = (
            lax.dot_general(
                s * (1.0 / DEG),
                w_ref[:, :D],
                dn,
                preferred_element_type=jnp.float32,
            )
            + lax.dot_general(
                m, w_ref[:, D:], dn, preferred_element_type=jnp.float32
            )
            + b_ref[...]
        )

    return fused_body


def _mm_body(a_ref, w_ref, b_ref, _aliased_ref, o_ref):
    dn = (((1,), (1,)), ((), ()))
    o_ref[...] = (
        lax.dot_general(
            a_ref[...], w_ref[...], dn, preferred_element_type=jnp.float32
        )
        + b_ref[...]
    )


def kernel(mailbox, W, b):
    N, DEG, D = mailbox.shape
    S = 2000
    C = 8
    Bn = 1000
    Bm = 1000

    b2 = b.reshape(1, D)

    agg = _make_agg_kernel(S, DEG, D, C)(mailbox)

    s_blocks = S // Bn
    out_tail = pl.pallas_call(
        _make_fused_body(DEG, D),
        grid=((N - S) // Bn,),
        in_specs=[
            pl.BlockSpec((Bn, DEG, D), lambda i: (i + s_blocks, 0, 0)),
            pl.BlockSpec((D, 2 * D), lambda i: (0, 0)),
            pl.BlockSpec((1, D), lambda i: (0, 0)),
        ],
        out_specs=pl.BlockSpec((Bn, D), lambda i: (i + s_blocks, 0)),
        out_shape=jax.ShapeDtypeStruct((N, D), jnp.float32),
    )(mailbox, W, b2)

    out = pl.pallas_call(
        _mm_body,
        grid=(S // Bm,),
        in_specs=[
            pl.BlockSpec((Bm, 2 * D), lambda i: (i, 0)),
            pl.BlockSpec((D, 2 * D), lambda i: (0, 0)),
            pl.BlockSpec((1, D), lambda i: (0, 0)),
            pl.BlockSpec(memory_space=pl.ANY),
        ],
        out_specs=pl.BlockSpec((Bm, D), lambda i: (i, 0)),
        out_shape=jax.ShapeDtypeStruct((N, D), jnp.float32),
        input_output_aliases={3: 0},
    )(agg, W, b2, out_tail)

    return out


# speedup vs baseline: 3.9959x; 1.3564x over previous
import functools

import jax
import jax.numpy as jnp
from jax import lax
from jax.experimental import pallas as pl
from jax.experimental.pallas import tpu as pltpu
from jax.experimental.pallas import tpu_sc as plsc

_NUM_CORES = 2
_NUM_SUBCORES = 16
_NW = _NUM_CORES * _NUM_SUBCORES
_LANES = 16


def _make_agg_kernel(S, DEG, D, C):
    assert S % C == 0
    n_chunks = S // C
    assert n_chunks >= 2 * _NW
    max_iters = (n_chunks + _NW - 1) // _NW
    half_iters = (max_iters + 1) // 2
    groups = D // _LANES
    inv_deg = 1.0 / DEG

    mesh = plsc.VectorSubcoreMesh(core_axis_name="c", subcore_axis_name="s")

    @functools.partial(
        pl.kernel,
        out_type=jax.ShapeDtypeStruct((S, 2 * D), jnp.float32),
        mesh=mesh,
        scratch_types=[
            pltpu.VMEM((2, C, DEG, D), jnp.float32),
            pltpu.VMEM((2, C, 2 * D), jnp.float32),
            pltpu.SemaphoreType.DMA,
            pltpu.SemaphoreType.DMA,
            pltpu.SemaphoreType.DMA,
            pltpu.SemaphoreType.DMA,
        ],
    )
    def agg(mb_hbm, out_hbm, buf, obuf, isem0, isem1, osem0, osem1):
        wid = lax.axis_index("s") * _NUM_CORES + lax.axis_index("c")
        my = (n_chunks - wid + _NW - 1) // _NW
        isems = (isem0, isem1)
        osems = (osem0, osem1)

        def in_copy(k, p):
            base = (k * _NW + wid) * C
            return pltpu.make_async_copy(
                mb_hbm.at[pl.ds(base, C)], buf.at[p], isems[p]
            )

        def out_copy(k, p):
            base = (k * _NW + wid) * C
            return pltpu.make_async_copy(
                obuf.at[p], out_hbm.at[pl.ds(base, C)], osems[p]
            )

        in_copy(0, 0).start()

        def body(kk, _):
            for p in range(2):
                k = kk * 2 + p
                np_ = 1 - p

                @pl.when(k + 1 < my)
                def _():
                    in_copy(k + 1, np_).start()

                @pl.when(k < my)
                def _():
                    in_copy(k, p).wait()

                    @pl.when(k >= 2)
                    def _():
                        out_copy(k - 2, p).wait()

                    def group_body(t, carry):
                        n = t // groups
                        g = t % groups
                        col = pl.ds(g * _LANES, _LANES)
                        s = buf[p, n, 0, col]
                        m = s
                        for r in range(1, DEG):
                            v = buf[p, n, r, col]
                            s = s + v
                            m = jnp.maximum(m, v)
                        obuf[p, n, col] = s * inv_deg
                        obuf[p, n, pl.ds(D + g * _LANES, _LANES)] = m
                        return carry

                    lax.fori_loop(0, C * groups, group_body, 0)
                    out_copy(k, p).start()

            return 0

        lax.fori_loop(0, half_iters, body, 0)

        out_copy(0, 0).wait()
        out_copy(0, 1).wait()

    return agg


def _make_fused_body(DEG, D):
    def fused_body(mb_ref, w_ref, b_ref, o_ref):
        blk = mb_ref[...]
        s = jnp.sum(blk, axis=1)
        m = jnp.max(blk, axis=1)
        dn = (((1,), (1,)), ((), ()))
        o_ref[...] = (
            lax.dot_general(
                s * (1.0 / DEG),
                w_ref[:, :D],
                dn,
                preferred_element_type=jnp.float32,
            )
            + lax.dot_general(
                m, w_ref[:, D:], dn, preferred_element_type=jnp.float32
            )
            + b_ref[...]
        )

    return fused_body


def _mm_body(a_ref, w_ref, b_ref, _aliased_ref, o_ref):
    dn = (((1,), (1,)), ((), ()))
    o_ref[...] = (
        lax.dot_general(
            a_ref[...], w_ref[...], dn, preferred_element_type=jnp.float32
        )
        + b_ref[...]
    )


def kernel(mailbox, W, b):
    N, DEG, D = mailbox.shape
    S = 0
    C = 8
    Bn = 1000
    Bm = 1000

    b2 = b.reshape(1, D)

    if S == 0:
        return pl.pallas_call(
            _make_fused_body(DEG, D),
            grid=(N // Bn,),
            in_specs=[
                pl.BlockSpec((Bn, DEG, D), lambda i: (i, 0, 0)),
                pl.BlockSpec((D, 2 * D), lambda i: (0, 0)),
                pl.BlockSpec((1, D), lambda i: (0, 0)),
            ],
            out_specs=pl.BlockSpec((Bn, D), lambda i: (i, 0)),
            out_shape=jax.ShapeDtypeStruct((N, D), jnp.float32),
        )(mailbox, W, b2)

    agg = _make_agg_kernel(S, DEG, D, C)(mailbox)

    s_blocks = S // Bn
    out_tail = pl.pallas_call(
        _make_fused_body(DEG, D),
        grid=((N - S) // Bn,),
        in_specs=[
            pl.BlockSpec((Bn, DEG, D), lambda i: (i + s_blocks, 0, 0)),
            pl.BlockSpec((D, 2 * D), lambda i: (0, 0)),
            pl.BlockSpec((1, D), lambda i: (0, 0)),
        ],
        out_specs=pl.BlockSpec((Bn, D), lambda i: (i + s_blocks, 0)),
        out_shape=jax.ShapeDtypeStruct((N, D), jnp.float32),
    )(mailbox, W, b2)

    out = pl.pallas_call(
        _mm_body,
        grid=(S // Bm,),
        in_specs=[
            pl.BlockSpec((Bm, 2 * D), lambda i: (i, 0)),
            pl.BlockSpec((D, 2 * D), lambda i: (0, 0)),
            pl.BlockSpec((1, D), lambda i: (0, 0)),
            pl.BlockSpec(memory_space=pl.ANY),
        ],
        out_specs=pl.BlockSpec((Bm, D), lambda i: (i, 0)),
        out_shape=jax.ShapeDtypeStruct((N, D), jnp.float32),
        input_output_aliases={3: 0},
    )(agg, W, b2, out_tail)

    return out
